# Initial kernel scaffold; baseline (speedup 1.0000x reference)
import jax
import jax.numpy as jnp
from jax.experimental import pallas as pl


def kernel(x_licitacion, x_empresa, edge_participa_src, edge_participa_dst, edge_rev_participa_src, edge_rev_participa_dst, W_l_p, W_r_p, b_p, W_l_r, W_r_r, b_r):
    raise NotImplementedError("write your pallas kernel here")


# speedup vs baseline: 3.6239x; 3.6239x over previous
import functools

import jax
import jax.numpy as jnp
from jax import lax
from jax.experimental import pallas as pl
from jax.experimental.pallas import tpu as pltpu
from jax.experimental.pallas import tpu_sc as plsc

N_LIC = 10000
N_EMP = 50000
D = 128
H = 128
E = 250000

NC = 2
NS = 16
NW = NC * NS

T_E = 8192
E_PAD = T_E * NW
G = 256
HALF = T_E // 2
N_CHUNKS = HALF // G
SCAT = G // 128
ROWS_PER_TILE = 632
N_PAD = ROWS_PER_TILE * NS

R_BLK = 1000


def _sc_agg_body(x_emp, src_hbm, dst2d_hbm, acc_out, cnt0_out, cnt1_out,
                 src_all, dst2d, rows_v, ones_v, zc, acc_sh, cnt_sh, gsem):
    c = lax.axis_index("c")
    s = lax.axis_index("s")
    wid = c * NS + s
    rbase = s * ROWS_PER_TILE

    @pl.loop(0, G)
    def _zero_rows(i):
        for j in range(D // 16):
            rows_v[i, pl.ds(j * 16, 16)] = jnp.zeros((16,), jnp.float32)

    @pl.loop(0, ROWS_PER_TILE // 16)
    def _zero_zc(i):
        zc[pl.ds(i * 16, 16)] = jnp.zeros((16,), jnp.float32)

    for j in range(128 // 16):
        ones_v[pl.ds(j * 16, 16)] = jnp.full((16,), 1.0, jnp.float32)

    pltpu.sync_copy(rows_v, acc_sh.at[pl.ds(rbase, G)])
    pltpu.sync_copy(rows_v, acc_sh.at[pl.ds(rbase + G, G)])
    pltpu.sync_copy(rows_v.at[pl.ds(0, ROWS_PER_TILE - 2 * G)],
                    acc_sh.at[pl.ds(rbase + 2 * G, ROWS_PER_TILE - 2 * G)])
    pltpu.sync_copy(zc, cnt_sh.at[pl.ds(rbase, ROWS_PER_TILE)])
    plsc.subcore_barrier()

    for h in range(2):
        pltpu.sync_copy(src_hbm.at[pl.ds(wid * T_E + h * HALF, HALF)], src_all)
        pltpu.sync_copy(
            dst2d_hbm.at[pl.ds(wid * (T_E // 128) + h * (HALF // 128),
                               HALF // 128)], dst2d)

        @pl.loop(0, N_CHUNKS)
        def _chunk(k):
            pltpu.async_copy(x_emp.at[src_all.at[pl.ds(k * G, G)]], rows_v,
                             gsem).wait()
            for j in range(SCAT):
                ridx = dst2d.at[k * SCAT + j]
                pltpu.sync_copy(rows_v.at[pl.ds(j * 128, 128)],
                                acc_sh.at[ridx], add=True)
                pltpu.sync_copy(ones_v, cnt_sh.at[ridx], add=True)

    plsc.subcore_barrier()

    pltpu.sync_copy(acc_sh.at[pl.ds(rbase, ROWS_PER_TILE)],
                    acc_out.at[c, pl.ds(rbase, ROWS_PER_TILE)])

    pltpu.sync_copy(cnt_sh.at[pl.ds(rbase, ROWS_PER_TILE)], zc)

    @pl.when(c == 0)
    def _():
        pltpu.sync_copy(zc, cnt0_out.at[pl.ds(rbase, ROWS_PER_TILE)])

    @pl.when(c == 1)
    def _():
        pltpu.sync_copy(zc, cnt1_out.at[pl.ds(rbase, ROWS_PER_TILE)])


def _sc_agg(x_emp, src_p, dst2d):
    mesh = plsc.VectorSubcoreMesh(core_axis_name="c", subcore_axis_name="s",
                                  num_cores=NC, num_subcores=NS)
    fn = pl.kernel(
        _sc_agg_body,
        out_type=[
            jax.ShapeDtypeStruct((NC, N_PAD, D), jnp.float32),
            jax.ShapeDtypeStruct((N_PAD,), jnp.float32),
            jax.ShapeDtypeStruct((N_PAD,), jnp.float32),
        ],
        mesh=mesh,
        scratch_types=[
            pltpu.VMEM((HALF,), jnp.int32),
            pltpu.VMEM((HALF // 128, 128), jnp.int32),
            pltpu.VMEM((G, D), jnp.float32),
            pltpu.VMEM((128,), jnp.float32),
            pltpu.VMEM((ROWS_PER_TILE,), jnp.float32),
            pltpu.VMEM_SHARED((N_PAD, D), jnp.float32),
            pltpu.VMEM_SHARED((N_PAD,), jnp.float32),
            pltpu.SemaphoreType.DMA,
        ],
    )
    return fn(x_emp, src_p, dst2d)


def _tc_body(acc_ref, cnt0_ref, cnt1_ref, x_ref, wl_ref, wr_ref, b_ref, out_ref):
    acc = acc_ref[0] + acc_ref[1]
    cnt = cnt0_ref[...] + cnt1_ref[...]
    mean = acc / jnp.maximum(cnt, 1.0)
    out = (jnp.dot(mean, wl_ref[...], preferred_element_type=jnp.float32)
           + b_ref[...]
           + jnp.dot(x_ref[...], wr_ref[...], preferred_element_type=jnp.float32))
    out_ref[...] = jnp.maximum(out, 0.0)


def _tc_combine(acc, cnt0, cnt1, x_lic, wl, wr, b2):
    return pl.pallas_call(
        _tc_body,
        grid=(N_LIC // R_BLK,),
        in_specs=[
            pl.BlockSpec((NC, R_BLK, D), lambda i: (0, i, 0)),
            pl.BlockSpec((R_BLK, 1), lambda i: (i, 0)),
            pl.BlockSpec((R_BLK, 1), lambda i: (i, 0)),
            pl.BlockSpec((R_BLK, D), lambda i: (i, 0)),
            pl.BlockSpec((D, H), lambda i: (0, 0)),
            pl.BlockSpec((D, H), lambda i: (0, 0)),
            pl.BlockSpec((1, H), lambda i: (0, 0)),
        ],
        out_specs=pl.BlockSpec((R_BLK, H), lambda i: (i, 0)),
        out_shape=jax.ShapeDtypeStruct((N_LIC, H), jnp.float32),
    )(acc, cnt0, cnt1, x_lic, wl, wr, b2)


def kernel(x_licitacion, x_empresa, edge_participa_src, edge_participa_dst,
           edge_rev_participa_src, edge_rev_participa_dst,
           W_l_p, W_r_p, b_p, W_l_r, W_r_r, b_r):
    src = edge_participa_src.astype(jnp.int32)
    dst = edge_participa_dst.astype(jnp.int32)
    src_p = jnp.concatenate([src, jnp.zeros((E_PAD - E,), jnp.int32)])
    dst_p = jnp.concatenate([dst, jnp.full((E_PAD - E,), N_LIC, jnp.int32)])
    dst2d = dst_p.reshape(E_PAD // 128, 128)

    acc, cnt0, cnt1 = _sc_agg(x_empresa, src_p, dst2d)
    return _tc_combine(acc, cnt0.reshape(N_PAD, 1), cnt1.reshape(N_PAD, 1),
                       x_licitacion, W_l_p, W_r_p, b_p.reshape(1, H))


# speedup vs baseline: 3.7623x; 1.0382x over previous
import functools

import jax
import jax.numpy as jnp
from jax import lax
from jax.experimental import pallas as pl
from jax.experimental.pallas import tpu as pltpu
from jax.experimental.pallas import tpu_sc as plsc

N_LIC = 10000
N_EMP = 50000
D = 128
H = 128
E = 250000

NC = 2
NS = 16
NW = NC * NS

T_E = 8192
E_PAD = T_E * NW
G = 128
HALF = T_E // 2
N_CHUNKS = HALF // G
N_ITERS = N_CHUNKS // 2
ROWS_PER_TILE = 632
N_PAD = ROWS_PER_TILE * NS

R_BLK = 1000


def _sc_agg_body(x_emp, src_hbm, dst2d_hbm, acc_out, cnt0_out, cnt1_out,
                 src_all, dst2d, b0, b1, ones_v, zc, acc_sh, cnt_sh,
                 g0, g1, s0, s1, c0, c1):
    c = lax.axis_index("c")
    s = lax.axis_index("s")
    wid = c * NS + s
    rbase = s * ROWS_PER_TILE

    @pl.loop(0, G)
    def _zero_rows(i):
        for j in range(D // 16):
            b0[i, pl.ds(j * 16, 16)] = jnp.zeros((16,), jnp.float32)
            b1[i, pl.ds(j * 16, 16)] = jnp.zeros((16,), jnp.float32)

    @pl.loop(0, ROWS_PER_TILE // 16)
    def _zero_zc(i):
        zc[pl.ds(i * 16, 16)] = jnp.zeros((16,), jnp.float32)

    for j in range(128 // 16):
        ones_v[pl.ds(j * 16, 16)] = jnp.full((16,), 1.0, jnp.float32)

    for q in range(4):
        pltpu.sync_copy(b0, acc_sh.at[pl.ds(rbase + q * G, G)])
    pltpu.sync_copy(b0.at[pl.ds(0, ROWS_PER_TILE - 4 * G)],
                    acc_sh.at[pl.ds(rbase + 4 * G, ROWS_PER_TILE - 4 * G)])
    pltpu.sync_copy(zc, cnt_sh.at[pl.ds(rbase, ROWS_PER_TILE)])
    plsc.subcore_barrier()

    def _gather(chunk, buf, sem):
        return pltpu.async_copy(
            x_emp.at[src_all.at[pl.ds(chunk * G, G)]], buf, sem)

    def _scat(chunk, buf, sem_a, sem_c):
        pltpu.async_copy(buf, acc_sh.at[dst2d.at[chunk]], sem_a, add=True)
        pltpu.async_copy(ones_v, cnt_sh.at[dst2d.at[chunk]], sem_c, add=True)

    def _scat_wait(buf, sem_a, sem_c):
        pltpu.make_async_copy(buf, acc_sh.at[dst2d.at[0]], sem_a).wait()
        pltpu.make_async_copy(ones_v, cnt_sh.at[dst2d.at[0]], sem_c).wait()

    for h in range(2):
        pltpu.sync_copy(src_hbm.at[pl.ds(wid * T_E + h * HALF, HALF)], src_all)
        pltpu.sync_copy(
            dst2d_hbm.at[pl.ds(wid * (T_E // 128) + h * (HALF // 128),
                               HALF // 128)], dst2d)
        _gather(0, b0, g0)

        @pl.loop(0, N_ITERS)
        def _pipe(i):
            j0 = 2 * i
            j1 = 2 * i + 1

            @pl.when(i > 0)
            def _():
                _scat_wait(b1, s1, c1)

            _gather(j1, b1, g1)
            pltpu.make_async_copy(
                x_emp.at[src_all.at[pl.ds(j0 * G, G)]], b0, g0).wait()
            _scat(j0, b0, s0, c0)
            _scat_wait(b0, s0, c0)

            @pl.when(i < N_ITERS - 1)
            def _():
                _gather(j0 + 2, b0, g0)

            pltpu.make_async_copy(
                x_emp.at[src_all.at[pl.ds(j1 * G, G)]], b1, g1).wait()
            _scat(j1, b1, s1, c1)

        _scat_wait(b1, s1, c1)

    plsc.subcore_barrier()

    pltpu.sync_copy(acc_sh.at[pl.ds(rbase, ROWS_PER_TILE)],
                    acc_out.at[c, pl.ds(rbase, ROWS_PER_TILE)])

    pltpu.sync_copy(cnt_sh.at[pl.ds(rbase, ROWS_PER_TILE)], zc)

    @pl.when(c == 0)
    def _():
        pltpu.sync_copy(zc, cnt0_out.at[pl.ds(rbase, ROWS_PER_TILE)])

    @pl.when(c == 1)
    def _():
        pltpu.sync_copy(zc, cnt1_out.at[pl.ds(rbase, ROWS_PER_TILE)])


def _sc_agg(x_emp, src_p, dst2d):
    mesh = plsc.VectorSubcoreMesh(core_axis_name="c", subcore_axis_name="s",
                                  num_cores=NC, num_subcores=NS)
    fn = pl.kernel(
        _sc_agg_body,
        out_type=[
            jax.ShapeDtypeStruct((NC, N_PAD, D), jnp.float32),
            jax.ShapeDtypeStruct((N_PAD,), jnp.float32),
            jax.ShapeDtypeStruct((N_PAD,), jnp.float32),
        ],
        mesh=mesh,
        scratch_types=[
            pltpu.VMEM((HALF,), jnp.int32),
            pltpu.VMEM((HALF // 128, 128), jnp.int32),
            pltpu.VMEM((G, D), jnp.float32),
            pltpu.VMEM((G, D), jnp.float32),
            pltpu.VMEM((128,), jnp.float32),
            pltpu.VMEM((ROWS_PER_TILE,), jnp.float32),
            pltpu.VMEM_SHARED((N_PAD, D), jnp.float32),
            pltpu.VMEM_SHARED((N_PAD,), jnp.float32),
            pltpu.SemaphoreType.DMA,
            pltpu.SemaphoreType.DMA,
            pltpu.SemaphoreType.DMA,
            pltpu.SemaphoreType.DMA,
            pltpu.SemaphoreType.DMA,
            pltpu.SemaphoreType.DMA,
        ],
    )
    return fn(x_emp, src_p, dst2d)


def _tc_body(acc_ref, cnt0_ref, cnt1_ref, x_ref, wl_ref, wr_ref, b_ref, out_ref):
    acc = acc_ref[0] + acc_ref[1]
    cnt = cnt0_ref[...] + cnt1_ref[...]
    mean = acc / jnp.maximum(cnt, 1.0)
    out = (jnp.dot(mean, wl_ref[...], preferred_element_type=jnp.float32)
           + b_ref[...]
           + jnp.dot(x_ref[...], wr_ref[...], preferred_element_type=jnp.float32))
    out_ref[...] = jnp.maximum(out, 0.0)


def _tc_combine(acc, cnt0, cnt1, x_lic, wl, wr, b2):
    return pl.pallas_call(
        _tc_body,
        grid=(N_LIC // R_BLK,),
        in_specs=[
            pl.BlockSpec((NC, R_BLK, D), lambda i: (0, i, 0)),
            pl.BlockSpec((R_BLK, 1), lambda i: (i, 0)),
            pl.BlockSpec((R_BLK, 1), lambda i: (i, 0)),
            pl.BlockSpec((R_BLK, D), lambda i: (i, 0)),
            pl.BlockSpec((D, H), lambda i: (0, 0)),
            pl.BlockSpec((D, H), lambda i: (0, 0)),
            pl.BlockSpec((1, H), lambda i: (0, 0)),
        ],
        out_specs=pl.BlockSpec((R_BLK, H), lambda i: (i, 0)),
        out_shape=jax.ShapeDtypeStruct((N_LIC, H), jnp.float32),
    )(acc, cnt0, cnt1, x_lic, wl, wr, b2)


def kernel(x_licitacion, x_empresa, edge_participa_src, edge_participa_dst,
           edge_rev_participa_src, edge_rev_participa_dst,
           W_l_p, W_r_p, b_p, W_l_r, W_r_r, b_r):
    src = edge_participa_src.astype(jnp.int32)
    dst = edge_participa_dst.astype(jnp.int32)
    src_p = jnp.concatenate([src, jnp.zeros((E_PAD - E,), jnp.int32)])
    dst_p = jnp.concatenate([dst, jnp.full((E_PAD - E,), N_LIC, jnp.int32)])
    dst2d = dst_p.reshape(E_PAD // 128, 128)

    acc, cnt0, cnt1 = _sc_agg(x_empresa, src_p, dst2d)
    return _tc_combine(acc, cnt0.reshape(N_PAD, 1), cnt1.reshape(N_PAD, 1),
                       x_licitacion, W_l_p, W_r_p, b_p.reshape(1, H))
